# baseline (speedup 1.0000x reference)
import functools

import jax
import jax.numpy as jnp
from jax import lax
from jax.experimental import pallas as pl
from jax.experimental.pallas import tpu as pltpu
from jax.experimental.pallas import tpu_sc as plsc

D = 64
LANES = 16


def _pack_pairs_kernel(num_cls, n_workers, num_cores):
    n_blocks = (num_cls + 127) // 128
    num_pairs_padded = (n_blocks * 128) // 2
    base_blocks = n_blocks // n_workers
    rem = n_blocks % n_workers

    mesh = plsc.VectorSubcoreMesh(core_axis_name="c", subcore_axis_name="s")

    @functools.partial(
        pl.kernel,
        mesh=mesh,
        out_type=jax.ShapeDtypeStruct((num_pairs_padded, 128), jnp.float32),
        scratch_types=[
            pltpu.VMEM((3, D, 128), jnp.float32),
            pltpu.VMEM((3, D, 128), jnp.float32),
            pltpu.SemaphoreType.DMA,
            pltpu.SemaphoreType.DMA,
        ],
        compiler_params=pltpu.CompilerParams(
            use_tc_tiling_on_sc=True, disable_bounds_checks=True,
            needs_layout_passes=False),
    )
    def p1(tab_t_hbm, out_hbm, in_v, out_v, sem_i, sem_o):
        wid = lax.axis_index("s") * num_cores + lax.axis_index("c")
        count = base_blocks + jnp.where(wid < rem, 1, 0)
        start = wid * base_blocks + jnp.minimum(wid, rem)

        iota = lax.iota(jnp.int32, LANES)

        def load(j, slot):
            pltpu.async_copy(
                tab_t_hbm.at[:, pl.ds((start + j) * 128, 128)],
                in_v.at[slot], sem_i)

        def wait_load(slot):
            pltpu.make_async_copy(
                tab_t_hbm.at[:, pl.ds(0, 128)], in_v.at[slot], sem_i).wait()

        def store(j, slot):
            pltpu.async_copy(
                out_v.at[slot],
                out_hbm.at[pl.ds((start + j) * D, D)], sem_o)

        def wait_store(slot):
            pltpu.make_async_copy(
                out_v.at[slot], out_hbm.at[pl.ds(0, D)], sem_o).wait()

        rots = [((iota + s) & (LANES - 1)) for s in range(LANES)]

        def shuffle(slot):
            src = in_v.at[slot]
            dst = out_v.at[slot]
            rows_l = [iota + f0 for f0 in range(0, D, LANES)]

            def vblock(i, carry):
                v0 = i * LANES
                for s in range(LANES):
                    vs = v0 + rots[s]
                    prow = vs >> 1
                    pcol = (vs & 1) * D
                    for fi in range(D // LANES):
                        vec = plsc.load_gather(src, [rows_l[fi], vs])
                        plsc.store_scatter(dst, [prow, pcol + rows_l[fi]],
                                           vec)
                return carry

            lax.fori_loop(0, 128 // LANES, vblock, 0)

        load(0, 0)
        load(1, 1)

        def outer(i3, carry):
            for b in range(3):
                slot = b

                @pl.when(i3 * 3 + b < count)
                def _():
                    j = i3 * 3 + b

                    @pl.when(j + 2 < count)
                    def _():
                        load(j + 2, (b + 2) % 3)

                    wait_load(slot)

                    @pl.when(j >= 3)
                    def _():
                        wait_store(slot)

                    shuffle(slot)
                    store(j, slot)
            return carry

        lax.fori_loop(0, (base_blocks + 3) // 3, outer, 0)
        wait_store(0)
        wait_store(1)
        wait_store(2)

    return p1, num_pairs_padded


def _gather_kernel(batch, hist, num_pairs_padded, n_workers, num_cores):
    n_bb = batch // 128
    n_units = hist * n_bb
    units_per_w = n_units // n_workers
    assert units_per_w * n_workers == n_units

    mesh = plsc.VectorSubcoreMesh(core_axis_name="c", subcore_axis_name="s")

    @functools.partial(
        pl.kernel,
        mesh=mesh,
        out_type=jax.ShapeDtypeStruct((hist, D, batch), jnp.float32),
        scratch_types=[
            pltpu.VMEM((3, 128), jnp.int32),
            pltpu.VMEM((3, 128), jnp.int32),
            pltpu.VMEM((3, 128), jnp.int32),
            pltpu.VMEM((3, 128, 128), jnp.float32),
            pltpu.VMEM((3, D, 128), jnp.float32),
            pltpu.SemaphoreType.DMA,
            pltpu.SemaphoreType.DMA,
            pltpu.SemaphoreType.DMA,
        ],
        compiler_params=pltpu.CompilerParams(
            use_tc_tiling_on_sc=True, disable_bounds_checks=True,
            needs_layout_passes=False),
    )
    def p2(data_t_hbm, tab2_hbm, out_hbm, idx_v, idxp_v, half_v, gath_v,
           outst_v, sem_g, sem_o, sem_i):
        wid = lax.axis_index("s") * num_cores + lax.axis_index("c")
        u0 = wid * units_per_w

        iota = lax.iota(jnp.int32, LANES)

        def load_idx(u, slot):
            h = u // n_bb
            bb = u % n_bb
            pltpu.async_copy(
                data_t_hbm.at[h, pl.ds(bb * 128, 128)], idx_v.at[slot],
                sem_i)

        def prep(u, slot):
            pltpu.make_async_copy(
                data_t_hbm.at[0, pl.ds(0, 128)], idx_v.at[slot],
                sem_i).wait()
            for j0 in range(0, 128, LANES):
                v = idx_v[slot, pl.ds(j0, LANES)]
                idxp_v[slot, pl.ds(j0, LANES)] = v >> 1
                half_v[slot, pl.ds(j0, LANES)] = (v & 1) * D
            pltpu.async_copy(
                tab2_hbm.at[idxp_v.at[slot]], gath_v.at[slot], sem_g)

        def wait_gather(slot):
            pltpu.make_async_copy(
                tab2_hbm.at[idxp_v.at[slot]], gath_v.at[slot], sem_g).wait()

        def writeback(u, slot):
            h = u // n_bb
            bb = u % n_bb
            pltpu.async_copy(
                outst_v.at[slot],
                out_hbm.at[h, :, pl.ds(bb * 128, 128)], sem_o)

        def wait_writeback(slot):
            pltpu.make_async_copy(
                outst_v.at[slot], out_hbm.at[0, :, pl.ds(0, 128)],
                sem_o).wait()

        rots = [((iota + s) & (LANES - 1)) for s in range(LANES)]
        rows_l = [iota + f0 for f0 in range(0, D, LANES)]

        def shuffle(slot):
            src = gath_v.at[slot]
            dst = outst_v.at[slot]
            hv = half_v.at[slot]

            def jblock(i, carry):
                j0 = i * LANES
                for s in range(LANES):
                    js = j0 + rots[s]
                    hrot = plsc.load_gather(hv, [js])
                    for fi in range(D // LANES):
                        vec = plsc.load_gather(src, [js, hrot + rows_l[fi]])
                        plsc.store_scatter(dst, [rows_l[fi], js], vec)
                return carry

            lax.fori_loop(0, 128 // LANES, jblock, 0)

        load_idx(u0, 0)
        load_idx(u0 + 1, 1)
        load_idx(u0 + 2, 2)
        prep(u0, 0)
        prep(u0 + 1, 1)

        def outer(i3, carry):
            for b in range(3):
                i = i3 * 3 + b
                u = u0 + i
                slot = b

                @pl.when(i < units_per_w)
                def _():
                    @pl.when(i + 3 < units_per_w)
                    def _():
                        load_idx(u + 3, slot)

                    @pl.when(i + 2 < units_per_w)
                    def _():
                        prep(u + 2, (b + 2) % 3)

                    wait_gather(slot)

                    @pl.when(i >= 3)
                    def _():
                        wait_writeback(slot)

                    shuffle(slot)
                    writeback(u, slot)
            return carry

        lax.fori_loop(0, (units_per_w + 2) // 3, outer, 0)
        wait_writeback(0)
        wait_writeback(1)
        wait_writeback(2)

    return p2


def kernel(data, table):
    batch, hist = data.shape
    num_cls = table.shape[0]
    info = plsc.get_sparse_core_info()
    n_workers = info.num_cores * info.num_subcores

    table_t = jnp.transpose(table)
    data_t = jnp.transpose(data)

    p1, num_pairs_padded = _pack_pairs_kernel(num_cls, n_workers,
                                              info.num_cores)
    table2 = p1(table_t)

    p2 = _gather_kernel(batch, hist, num_pairs_padded, n_workers,
                        info.num_cores)
    out_t = p2(data_t, table2)
    return jnp.transpose(out_t, (2, 0, 1))


# speedup vs baseline: 1.0441x; 1.0441x over previous
import functools

import jax
import jax.numpy as jnp
from jax import lax
from jax.experimental import pallas as pl
from jax.experimental.pallas import tpu as pltpu
from jax.experimental.pallas import tpu_sc as plsc

D = 64
LANES = 16


def _pack_pairs_kernel(num_cls, n_workers, num_cores):
    n_blocks = (num_cls + 127) // 128
    num_pairs_padded = (n_blocks * 128) // 2
    base_blocks = n_blocks // n_workers
    rem = n_blocks % n_workers

    mesh = plsc.VectorSubcoreMesh(core_axis_name="c", subcore_axis_name="s")

    @functools.partial(
        pl.kernel,
        mesh=mesh,
        out_type=jax.ShapeDtypeStruct((num_pairs_padded, 128), jnp.float32),
        scratch_types=[
            pltpu.VMEM((2, D, 128), jnp.float32),
            pltpu.VMEM((2, D, 128), jnp.float32),
            pltpu.SemaphoreType.DMA,
            pltpu.SemaphoreType.DMA,
        ],
        compiler_params=pltpu.CompilerParams(
            use_tc_tiling_on_sc=True, disable_bounds_checks=True,
            needs_layout_passes=False),
    )
    def p1(tab_t_hbm, out_hbm, in_v, out_v, sem_i, sem_o):
        wid = lax.axis_index("s") * num_cores + lax.axis_index("c")
        count = base_blocks + jnp.where(wid < rem, 1, 0)
        start = wid * base_blocks + jnp.minimum(wid, rem)

        iota = lax.iota(jnp.int32, LANES)

        def load(j, slot):
            pltpu.async_copy(
                tab_t_hbm.at[:, pl.ds((start + j) * 128, 128)],
                in_v.at[slot], sem_i)

        def wait_load(slot):
            pltpu.make_async_copy(
                tab_t_hbm.at[:, pl.ds(0, 128)], in_v.at[slot], sem_i).wait()

        def store(j, slot):
            pltpu.async_copy(
                out_v.at[slot],
                out_hbm.at[pl.ds((start + j) * D, D)], sem_o)

        def wait_store(slot):
            pltpu.make_async_copy(
                out_v.at[slot], out_hbm.at[pl.ds(0, D)], sem_o).wait()

        rots = [((iota + s) & (LANES - 1)) for s in range(LANES)]

        def shuffle(slot):
            src = in_v.at[slot]
            dst = out_v.at[slot]
            rows_l = [iota + f0 for f0 in range(0, D, LANES)]

            def vblock(i, carry):
                v0 = i * LANES
                for s in range(LANES):
                    vs = v0 + rots[s]
                    prow = vs >> 1
                    pcol = (vs & 1) * D
                    for fi in range(D // LANES):
                        vec = plsc.load_gather(src, [rows_l[fi], vs])
                        plsc.store_scatter(dst, [prow, pcol + rows_l[fi]],
                                           vec)
                return carry

            lax.fori_loop(0, 128 // LANES, vblock, 0)

        load(0, 0)

        def outer(i2, carry):
            for b in range(2):
                slot = b

                @pl.when(i2 * 2 + b < count)
                def _():
                    j = i2 * 2 + b

                    @pl.when(j + 1 < count)
                    def _():
                        load(j + 1, 1 - slot)

                    wait_load(slot)

                    @pl.when(j >= 2)
                    def _():
                        wait_store(slot)

                    shuffle(slot)
                    store(j, slot)
            return carry

        lax.fori_loop(0, (base_blocks + 2) // 2, outer, 0)
        wait_store(0)
        wait_store(1)

    return p1, num_pairs_padded


def _gather_kernel(batch, hist, num_pairs_padded, n_workers, num_cores):
    n_bb = batch // 128
    n_units = hist * n_bb
    units_per_w = n_units // n_workers
    assert units_per_w * n_workers == n_units

    mesh = plsc.VectorSubcoreMesh(core_axis_name="c", subcore_axis_name="s")

    @functools.partial(
        pl.kernel,
        mesh=mesh,
        out_type=jax.ShapeDtypeStruct((hist, D, batch), jnp.float32),
        scratch_types=[
            pltpu.VMEM((2, 128), jnp.int32),
            pltpu.VMEM((2, 128), jnp.int32),
            pltpu.VMEM((2, 128), jnp.int32),
            pltpu.VMEM((2, 128, 128), jnp.float32),
            pltpu.VMEM((2, D, 128), jnp.float32),
            pltpu.SemaphoreType.DMA,
            pltpu.SemaphoreType.DMA,
            pltpu.SemaphoreType.DMA,
        ],
        compiler_params=pltpu.CompilerParams(
            use_tc_tiling_on_sc=True, disable_bounds_checks=True,
            needs_layout_passes=False),
    )
    def p2(data_t_hbm, tab2_hbm, out_hbm, idx_v, idxp_v, half_v, gath_v,
           outst_v, sem_g, sem_o, sem_i):
        wid = lax.axis_index("s") * num_cores + lax.axis_index("c")
        u0 = wid * units_per_w

        iota = lax.iota(jnp.int32, LANES)

        def load_idx(u, slot):
            h = u // n_bb
            bb = u % n_bb
            pltpu.async_copy(
                data_t_hbm.at[h, pl.ds(bb * 128, 128)], idx_v.at[slot],
                sem_i)

        def prep(u, slot):
            pltpu.make_async_copy(
                data_t_hbm.at[0, pl.ds(0, 128)], idx_v.at[slot],
                sem_i).wait()
            for j0 in range(0, 128, LANES):
                v = idx_v[slot, pl.ds(j0, LANES)]
                idxp_v[slot, pl.ds(j0, LANES)] = v >> 1
                half_v[slot, pl.ds(j0, LANES)] = (v & 1) * D
            pltpu.async_copy(
                tab2_hbm.at[idxp_v.at[slot]], gath_v.at[slot], sem_g)

        def wait_gather(slot):
            pltpu.make_async_copy(
                tab2_hbm.at[idxp_v.at[slot]], gath_v.at[slot], sem_g).wait()

        def writeback(u, slot):
            h = u // n_bb
            bb = u % n_bb
            pltpu.async_copy(
                outst_v.at[slot],
                out_hbm.at[h, :, pl.ds(bb * 128, 128)], sem_o)

        def wait_writeback(slot):
            pltpu.make_async_copy(
                outst_v.at[slot], out_hbm.at[0, :, pl.ds(0, 128)],
                sem_o).wait()

        rots = [((iota + s) & (LANES - 1)) for s in range(LANES)]
        rows_l = [iota + f0 for f0 in range(0, D, LANES)]

        def shuffle(slot):
            src = gath_v.at[slot]
            dst = outst_v.at[slot]
            hv = half_v.at[slot]

            def jblock(i, carry):
                j0 = i * LANES
                for s in range(LANES):
                    js = j0 + rots[s]
                    hrot = plsc.load_gather(hv, [js])
                    for fi in range(D // LANES):
                        vec = plsc.load_gather(src, [js, hrot + rows_l[fi]])
                        plsc.store_scatter(dst, [rows_l[fi], js], vec)
                return carry

            lax.fori_loop(0, 128 // LANES, jblock, 0)

        load_idx(u0, 0)
        load_idx(u0 + 1, 1)
        prep(u0, 0)

        def outer(i2, carry):
            for b in range(2):
                i = i2 * 2 + b
                u = u0 + i
                slot = b

                @pl.when(i + 2 < units_per_w)
                def _():
                    load_idx(u + 2, slot)

                @pl.when(i + 1 < units_per_w)
                def _():
                    prep(u + 1, 1 - slot)

                wait_gather(slot)

                @pl.when(i >= 2)
                def _():
                    wait_writeback(slot)

                shuffle(slot)
                writeback(u, slot)
            return carry

        lax.fori_loop(0, units_per_w // 2, outer, 0)
        wait_writeback(0)
        wait_writeback(1)

    return p2


def kernel(data, table):
    batch, hist = data.shape
    num_cls = table.shape[0]
    info = plsc.get_sparse_core_info()
    n_workers = info.num_cores * info.num_subcores

    table_t = jnp.transpose(table)
    data_t = jnp.transpose(data)

    p1, num_pairs_padded = _pack_pairs_kernel(num_cls, n_workers,
                                              info.num_cores)
    table2 = p1(table_t)

    p2 = _gather_kernel(batch, hist, num_pairs_padded, n_workers,
                        info.num_cores)
    out_t = p2(data_t, table2)
    return jnp.transpose(out_t, (2, 0, 1))


# speedup vs baseline: 1.0973x; 1.0510x over previous
import functools

import jax
import jax.numpy as jnp
from jax import lax
from jax.experimental import pallas as pl
from jax.experimental.pallas import tpu as pltpu
from jax.experimental.pallas import tpu_sc as plsc

D = 64
LANES = 16


def _pack_pairs_kernel(num_cls, n_workers, num_cores):
    n_blocks = (num_cls + 127) // 128
    num_pairs_padded = (n_blocks * 128) // 2
    base_blocks = n_blocks // n_workers
    rem = n_blocks % n_workers

    mesh = plsc.VectorSubcoreMesh(core_axis_name="c", subcore_axis_name="s")

    @functools.partial(
        pl.kernel,
        mesh=mesh,
        out_type=jax.ShapeDtypeStruct((num_pairs_padded, 128), jnp.float32),
        scratch_types=[
            pltpu.VMEM((2, D, 128), jnp.float32),
            pltpu.VMEM((2, D, 128), jnp.float32),
            pltpu.SemaphoreType.DMA,
            pltpu.SemaphoreType.DMA,
        ],
        compiler_params=pltpu.CompilerParams(
            use_tc_tiling_on_sc=True, disable_bounds_checks=True,
            needs_layout_passes=False),
    )
    def p1(tab_t_hbm, out_hbm, in_v, out_v, sem_i, sem_o):
        wid = lax.axis_index("s") * num_cores + lax.axis_index("c")
        count = base_blocks + jnp.where(wid < rem, 1, 0)
        start = wid * base_blocks + jnp.minimum(wid, rem)

        iota = lax.iota(jnp.int32, LANES)

        def load(j, slot):
            pltpu.async_copy(
                tab_t_hbm.at[:, pl.ds((start + j) * 128, 128)],
                in_v.at[slot], sem_i)

        def wait_load(slot):
            pltpu.make_async_copy(
                tab_t_hbm.at[:, pl.ds(0, 128)], in_v.at[slot], sem_i).wait()

        def store(j, slot):
            pltpu.async_copy(
                out_v.at[slot],
                out_hbm.at[pl.ds((start + j) * D, D)], sem_o)

        def wait_store(slot):
            pltpu.make_async_copy(
                out_v.at[slot], out_hbm.at[pl.ds(0, D)], sem_o).wait()

        rots = [((iota + s) & (LANES - 1)) for s in range(LANES)]

        def shuffle(slot):
            src = in_v.at[slot]
            dst = out_v.at[slot]
            rows_l = [iota + f0 for f0 in range(0, D, LANES)]

            def vblock(i, carry):
                v0 = i * LANES
                for s in range(LANES):
                    vs = v0 + rots[s]
                    prow = vs >> 1
                    pcol = (vs & 1) * D
                    for fi in range(D // LANES):
                        vec = plsc.load_gather(src, [rows_l[fi], vs])
                        plsc.store_scatter(dst, [prow, pcol + rows_l[fi]],
                                           vec)
                return carry

            lax.fori_loop(0, 128 // LANES, vblock, 0)

        load(0, 0)

        def outer(i2, carry):
            for b in range(2):
                slot = b

                @pl.when(i2 * 2 + b < count)
                def _():
                    j = i2 * 2 + b

                    @pl.when(j + 1 < count)
                    def _():
                        load(j + 1, 1 - slot)

                    wait_load(slot)

                    @pl.when(j >= 2)
                    def _():
                        wait_store(slot)

                    shuffle(slot)
                    store(j, slot)
            return carry

        lax.fori_loop(0, (base_blocks + 2) // 2, outer, 0)
        wait_store(0)
        wait_store(1)

    return p1, num_pairs_padded


def _gather_kernel(batch, hist, num_pairs_padded, n_workers, num_cores):
    WB = 256
    n_bb = batch // WB
    n_units = hist * n_bb
    units_per_w = n_units // n_workers
    assert units_per_w * n_workers == n_units

    mesh = plsc.VectorSubcoreMesh(core_axis_name="c", subcore_axis_name="s")

    @functools.partial(
        pl.kernel,
        mesh=mesh,
        out_type=jax.ShapeDtypeStruct((hist, D, batch), jnp.float32),
        scratch_types=[
            pltpu.VMEM((2, WB), jnp.int32),
            pltpu.VMEM((2, WB // 128, 128), jnp.int32),
            pltpu.VMEM((2, WB // 128, 128), jnp.int32),
            pltpu.VMEM((2, WB, 128), jnp.float32),
            pltpu.VMEM((2, D, WB), jnp.float32),
            pltpu.SemaphoreType.DMA,
            pltpu.SemaphoreType.DMA,
            pltpu.SemaphoreType.DMA,
        ],
        compiler_params=pltpu.CompilerParams(
            use_tc_tiling_on_sc=True, disable_bounds_checks=True,
            needs_layout_passes=False),
    )
    def p2(data_t_hbm, tab2_hbm, out_hbm, idx_v, idxp_v, half_v, gath_v,
           outst_v, sem_g, sem_o, sem_i):
        wid = lax.axis_index("s") * num_cores + lax.axis_index("c")
        u0 = wid * units_per_w

        iota = lax.iota(jnp.int32, LANES)

        def load_idx(u, slot):
            h = u // n_bb
            bb = u % n_bb
            pltpu.async_copy(
                data_t_hbm.at[h, pl.ds(bb * WB, WB)], idx_v.at[slot],
                sem_i)

        def prep(u, slot):
            pltpu.make_async_copy(
                data_t_hbm.at[0, pl.ds(0, WB)], idx_v.at[slot],
                sem_i).wait()
            for j0 in range(0, WB, LANES):
                v = idx_v[slot, pl.ds(j0, LANES)]
                idxp_v[slot, j0 // 128, pl.ds(j0 % 128, LANES)] = v >> 1
                half_v[slot, j0 // 128, pl.ds(j0 % 128, LANES)] = (v & 1) * D
            for k in range(WB // 128):
                pltpu.async_copy(
                    tab2_hbm.at[idxp_v.at[slot, k]],
                    gath_v.at[slot, pl.ds(k * 128, 128)], sem_g)

        def wait_gather(slot):
            for k in range(WB // 128):
                pltpu.make_async_copy(
                    tab2_hbm.at[idxp_v.at[slot, k]],
                    gath_v.at[slot, pl.ds(k * 128, 128)], sem_g).wait()

        def writeback(u, slot):
            h = u // n_bb
            bb = u % n_bb
            pltpu.async_copy(
                outst_v.at[slot],
                out_hbm.at[h, :, pl.ds(bb * WB, WB)], sem_o)

        def wait_writeback(slot):
            pltpu.make_async_copy(
                outst_v.at[slot], out_hbm.at[0, :, pl.ds(0, WB)],
                sem_o).wait()

        rots = [((iota + s) & (LANES - 1)) for s in range(LANES)]
        rows_l = [iota + f0 for f0 in range(0, D, LANES)]

        def shuffle(slot):
            src = gath_v.at[slot]
            dst = outst_v.at[slot]
            hv = half_v.at[slot]

            def jblock(i, carry):
                j0 = i * LANES
                for s in range(LANES):
                    js = j0 + rots[s]
                    hrot = plsc.load_gather(hv, [js >> 7, js & 127])
                    for fi in range(D // LANES):
                        vec = plsc.load_gather(src, [js, hrot + rows_l[fi]])
                        plsc.store_scatter(dst, [rows_l[fi], js], vec)
                return carry

            lax.fori_loop(0, WB // LANES, jblock, 0)

        load_idx(u0, 0)
        load_idx(u0 + 1, 1)
        prep(u0, 0)

        def outer(i2, carry):
            for b in range(2):
                i = i2 * 2 + b
                u = u0 + i
                slot = b

                @pl.when(i + 2 < units_per_w)
                def _():
                    load_idx(u + 2, slot)

                @pl.when(i + 1 < units_per_w)
                def _():
                    prep(u + 1, 1 - slot)

                wait_gather(slot)

                @pl.when(i >= 2)
                def _():
                    wait_writeback(slot)

                shuffle(slot)
                writeback(u, slot)
            return carry

        lax.fori_loop(0, units_per_w // 2, outer, 0)
        wait_writeback(0)
        wait_writeback(1)

    return p2


def kernel(data, table):
    batch, hist = data.shape
    num_cls = table.shape[0]
    info = plsc.get_sparse_core_info()
    n_workers = info.num_cores * info.num_subcores

    table_t = jnp.transpose(table)
    data_t = jnp.transpose(data)

    p1, num_pairs_padded = _pack_pairs_kernel(num_cls, n_workers,
                                              info.num_cores)
    table2 = p1(table_t)

    p2 = _gather_kernel(batch, hist, num_pairs_padded, n_workers,
                        info.num_cores)
    out_t = p2(data_t, table2)
    return jnp.transpose(out_t, (2, 0, 1))
